# Initial kernel scaffold; baseline (speedup 1.0000x reference)
import jax
import jax.numpy as jnp
from jax.experimental import pallas as pl


def kernel(x, emb):
    raise NotImplementedError("write your pallas kernel here")


# speedup vs baseline: 1.8759x; 1.8759x over previous
import functools

import jax
import jax.numpy as jnp
from jax import lax
from jax.experimental import pallas as pl
from jax.experimental.pallas import tpu as pltpu
from jax.experimental.pallas import tpu_sc as plsc

VOCAB = 1_000_000
DIM = 64

NC = 2
NS = 16
NW = NC * NS

B_TOTAL = 16384 * 50
BPW = B_TOTAL // NW
CHUNK = 128
K = BPW // CHUNK
NBUF = 8
OUTER = K // NBUF

_mesh = plsc.VectorSubcoreMesh(
    core_axis_name="c", subcore_axis_name="s", num_cores=NC, num_subcores=NS
)


@functools.partial(
    pl.kernel,
    out_type=jax.ShapeDtypeStruct((B_TOTAL, DIM), jnp.float32),
    mesh=_mesh,
    scratch_types=[
        pltpu.VMEM((K, CHUNK), jnp.int32),
        pltpu.VMEM((NBUF, CHUNK, DIM), jnp.float32),
        [pltpu.SemaphoreType.DMA] * NBUF,
    ],
    compiler_params=pltpu.CompilerParams(use_tc_tiling_on_sc=False),
)
def _emb_lookup(idx_hbm, table_hbm, out_hbm, idx_v, rows_v, gsems):
    wid = lax.axis_index("s") * NC + lax.axis_index("c")
    pltpu.sync_copy(idx_hbm.at[wid], idx_v)
    base = wid * BPW

    def fire(kk, b):
        pltpu.async_copy(table_hbm.at[idx_v.at[kk]], rows_v.at[b], gsems[b])

    def drain(kk, b):
        pltpu.make_async_copy(table_hbm.at[idx_v.at[kk]], rows_v.at[b],
                              gsems[b]).wait()
        pltpu.sync_copy(rows_v.at[b], out_hbm.at[pl.ds(base + kk * CHUNK, CHUNK)])

    for b in range(NBUF):
        fire(b, b)

    @pl.loop(0, OUTER - 1)
    def _outer(o):
        for b in range(NBUF):
            kk = o * NBUF + b
            drain(kk, b)
            fire(kk + NBUF, b)

    for b in range(NBUF):
        drain((OUTER - 1) * NBUF + b, b)


def kernel(x, emb):
    idx = x.reshape(NW, K, CHUNK)
    out = _emb_lookup(idx, emb)
    return out.reshape(x.shape + (DIM,))
